# baseline (speedup 1.0000x reference)
import functools

import jax
import jax.numpy as jnp
from jax.experimental import pallas as pl
from jax.experimental.pallas import tpu as pltpu
from jax.experimental.pallas import tpu_sc as plsc

_PAD = 0
_W = 128
_GP = 2
_NB = 3
_NC = 2
_NS = 16


def _gather_sc(table, idx2d):
    n_win, w = idx2d.shape
    emb = table.shape[1]
    n_idx = n_win * w
    nw = _NC * _NS
    n = n_win // (nw * _GP)
    pr = _GP * w
    mesh = plsc.VectorSubcoreMesh(core_axis_name="core",
                                  subcore_axis_name="subcore")

    @functools.partial(
        pl.kernel,
        out_type=jax.ShapeDtypeStruct((n_idx, emb), table.dtype),
        mesh=mesh,
        scratch_types=(
            [pltpu.VMEM((n * _GP, w), jnp.int32),
             pltpu.VMEM((_NB * pr, emb), table.dtype)]
            + [pltpu.SemaphoreType.DMA for _ in range(2 * _NB)]
        ),
    )
    def k(x_hbm, i_hbm, o_hbm, idx_v, buf, *sems):
        gsems = sems[:_NB]
        ssems = sems[_NB:]
        wid = jax.lax.axis_index("subcore") * _NC + jax.lax.axis_index("core")
        row0 = wid * n * pr

        pltpu.sync_copy(i_hbm.at[pl.ds(wid * n * _GP, n * _GP)], idx_v)

        def issue_gathers(s, p):
            for h in range(_GP):
                pltpu.async_copy(x_hbm.at[idx_v.at[s * _GP + h]],
                                 buf.at[pl.ds(p * pr + h * w, w)], gsems[p])

        def wait_gathers(p):
            for h in range(_GP):
                pltpu.make_async_copy(x_hbm.at[idx_v.at[0]],
                                     buf.at[pl.ds(p * pr + h * w, w)],
                                     gsems[p]).wait()

        def issue_store(s, p):
            pltpu.async_copy(buf.at[pl.ds(p * pr, pr)],
                             o_hbm.at[pl.ds(row0 + s * pr, pr)], ssems[p])

        def wait_store(p):
            pltpu.make_async_copy(buf.at[pl.ds(p * pr, pr)],
                                  o_hbm.at[pl.ds(0, pr)], ssems[p]).wait()

        for s in range(_NB - 1):
            issue_gathers(s, s % _NB)

        for s in range(_NB):
            wait_gathers(s % _NB)
            issue_store(s, s % _NB)
            p2 = (s + _NB - 1) % _NB
            if s >= 1:
                wait_store(p2)
            if s + _NB - 1 < n:
                issue_gathers(s + _NB - 1, p2)

        n_main = ((n - (_NB - 1) - _NB) // _NB) * _NB

        @pl.loop(0, n_main, step=_NB)
        def _(t):
            for b in range(_NB):
                s = _NB + t + b
                wait_gathers(b)
                issue_store(s, b)
                p2 = (b + _NB - 1) % _NB
                wait_store(p2)
                issue_gathers(s + _NB - 1, p2)

        for s in range(_NB + n_main, n):
            wait_gathers(s % _NB)
            issue_store(s, s % _NB)
            if s + _NB - 1 < n:
                wait_store((s + _NB - 1) % _NB)
                issue_gathers(s + _NB - 1, (s + _NB - 1) % _NB)

        for p in range(_NB):
            wait_store(p)

    return k(table, idx2d)


def _mask_len_tc(inputs):
    b, l = inputs.shape

    def body(x_ref, m_ref, len_ref):
        x = x_ref[...]
        m = x != _PAD
        m_ref[...] = m
        len_ref[...] = jnp.sum(m.astype(jnp.int32), axis=1, keepdims=True)

    return pl.pallas_call(
        body,
        out_shape=(jax.ShapeDtypeStruct((b, l), jnp.bool_),
                   jax.ShapeDtypeStruct((b, 1), jnp.int32)),
    )(inputs)


def kernel(table, inputs):
    b, l = inputs.shape
    emb = table.shape[1]
    idx2d = inputs.reshape(b * l // _W, _W)
    emb_flat = _gather_sc(table, idx2d)
    masks, lengths = _mask_len_tc(inputs)
    return emb_flat.reshape(b, l, emb), lengths.reshape(b), masks
